# baseline (speedup 1.0000x reference)
import functools

import jax
import jax.numpy as jnp
from jax import lax
from jax.experimental import pallas as pl
from jax.experimental.pallas import tpu as pltpu
from jax.experimental.pallas import tpu_sc as plsc

D = 64
NC = 2
NS = 16
NW = NC * NS
N_NODES = 1000000
CW = 512
N_FULL = N_NODES // CW
TAIL_LO = N_FULL * CW
TAIL_W = N_NODES - TAIL_LO
CHUNKS_BASE = N_FULL // NW
CHUNKS_EXTRA = N_FULL - CHUNKS_BASE * NW
IDX_PIECE = 8192
MAXJJ = (CHUNKS_BASE + CHUNKS_EXTRA + 2) // 3
BKT_CAP = 2080
CH_CAP = 560


def _iota16():
    return lax.iota(jnp.int32, 16)


def _in_range(vec, lo, hi):
    ge = lax.min(lax.max(vec - lo + 1, 0), 1)
    lt = lax.min(lax.max(hi - vec, 0), 1)
    return ge * lt


def _compact_append(ref_a, ref_b, va, vb, mi, cnt, trash):
    pos = plsc.cumsum(mi)
    dst = (cnt + pos - 1) * mi + (trash + _iota16()) * (1 - mi)
    plsc.store_scatter(ref_a, [dst], va)
    plsc.store_scatter(ref_b, [dst], vb)
    return cnt + pos[15]


def _sc_scan_fn(total):
    n_vec = total // 16

    def body(tbl, idx_hbm, out_hbm, idx_v, buf0, buf1, buf2, tail_v,
             bkt_r, bkt_s, chr_v, chs_v, stage, hg_s, semc, semo):
        wid = lax.axis_index("s") * NC + lax.axis_index("c")
        is0 = 1 - lax.min(wid, 1)
        nch = CHUNKS_BASE + CHUNKS_EXTRA * is0
        ch0 = wid * CHUNKS_BASE + CHUNKS_EXTRA * (1 - is0)
        c_lo = ch0 * CW
        c_hi = c_lo + nch * CW + TAIL_W * lax.max(wid - (NW - 2), 0)

        def fire_chunk(lo, buf):
            for c8 in range(8):
                pltpu.async_copy(
                    tbl.at[pl.ds(c8 * 8, 8), pl.ds(lo, CW)],
                    buf.at[pl.ds(c8 * 8, 8), :], semc)

        def wait_chunk(lo, buf):
            for c8 in range(8):
                pltpu.make_async_copy(
                    tbl.at[pl.ds(c8 * 8, 8), pl.ds(lo, CW)],
                    buf.at[pl.ds(c8 * 8, 8), :], semc).wait()

        fire_chunk(c_lo, buf0)
        fire_chunk(c_lo + CW, buf1)
        fire_chunk(c_lo + 2 * CW, buf2)

        hg_s[0] = 0

        cnt0 = jnp.int32(0)
        for piece in range(total // IDX_PIECE):
            pltpu.sync_copy(idx_hbm.at[pl.ds(piece * IDX_PIECE, IDX_PIECE)],
                            idx_v)

            def p1(g, cnt, _piece=piece):
                vec = idx_v[pl.ds(g * 16, 16)]
                mi = _in_range(vec, c_lo, c_hi)
                slots = _iota16() + (_piece * IDX_PIECE + g * 16)
                return _compact_append(bkt_r, bkt_s, vec, slots, mi, cnt,
                                       BKT_CAP - 16)

            cnt0 = lax.fori_loop(0, IDX_PIECE // 16, p1, cnt0)
        nb = cnt0
        nbv = lax.shift_right_logical(nb + 15, 4)

        def extract_hits(lo, buf, ccnt):
            def exhit(i, carry):
                r = chr_v[pl.ds(i, 16)][0]
                slot = chs_v[pl.ds(i, 16)][0]
                rr = r - lo
                hg = hg_s[0]
                ring = lax.bitwise_and(hg, jnp.int32(15))

                @pl.when(hg >= 16)
                def _():
                    pltpu.make_async_copy(
                        out_hbm.at[pl.ds(0, 128)],
                        stage.at[pl.ds(0, 128)], semo).wait()

                for grp in range(4):
                    val = plsc.load_gather(
                        buf, [_iota16() + grp * 16,
                              jnp.broadcast_to(rr, (16,))])
                    stage[pl.ds(ring * 128 + grp * 16, 16)] = val
                pltpu.async_copy(stage.at[pl.ds(ring * 128, 128)],
                                 out_hbm.at[pl.ds(slot * 128, 128)], semo)
                hg_s[0] = hg + 1
                return carry

            lax.fori_loop(0, ccnt, exhit, jnp.int32(0))

        def scan_bucket(lo, width):
            def sv(v, ccnt):
                vecr = bkt_r[pl.ds(v * 16, 16)]
                vecs = bkt_s[pl.ds(v * 16, 16)]
                mi = _in_range(vecr, lo, lo + width)
                mi = mi * _in_range(_iota16() + v * 16, 0, nb)
                return _compact_append(chr_v, chs_v, vecr, vecs, mi, ccnt,
                                       CH_CAP - 16)

            return lax.fori_loop(0, nbv, sv, jnp.int32(0))

        def process(j, buf):
            lo = c_lo + j * CW
            wait_chunk(lo, buf)
            ccnt = scan_bucket(lo, CW)
            extract_hits(lo, buf, ccnt)

            @pl.when(j + 3 < nch)
            def _():
                fire_chunk(lo + 3 * CW, buf)

        def jj_body(jj, carry):
            j0 = jj * 3

            @pl.when(j0 < nch)
            def _():
                process(j0, buf0)

            @pl.when(j0 + 1 < nch)
            def _():
                process(j0 + 1, buf1)

            @pl.when(j0 + 2 < nch)
            def _():
                process(j0 + 2, buf2)

            return carry

        lax.fori_loop(0, MAXJJ, jj_body, jnp.int32(0))

        @pl.when(wid == NW - 1)
        def _():
            pltpu.sync_copy(tbl.at[:, pl.ds(TAIL_LO, TAIL_W)], tail_v)
            ccnt = scan_bucket(jnp.int32(TAIL_LO), jnp.int32(TAIL_W))
            extract_hits(jnp.int32(TAIL_LO), tail_v, ccnt)

        def drain(i, carry):
            pltpu.make_async_copy(out_hbm.at[pl.ds(0, 128)],
                                  stage.at[pl.ds(0, 128)], semo).wait()
            return carry

        lax.fori_loop(0, lax.min(hg_s[0], jnp.int32(16)), drain, jnp.int32(0))

    return body


@functools.lru_cache(maxsize=None)
def _make_sc_scan(total):
    mesh = plsc.VectorSubcoreMesh(core_axis_name="c", subcore_axis_name="s")
    return pl.kernel(
        _sc_scan_fn(total),
        mesh=mesh,
        out_type=jax.ShapeDtypeStruct((total * 128,), jnp.float32),
        scratch_types=[
            pltpu.VMEM((IDX_PIECE,), jnp.int32),
            pltpu.VMEM((D, CW), jnp.float32),
            pltpu.VMEM((D, CW), jnp.float32),
            pltpu.VMEM((D, CW), jnp.float32),
            pltpu.VMEM((D, TAIL_W), jnp.float32),
            pltpu.VMEM((BKT_CAP,), jnp.int32),
            pltpu.VMEM((BKT_CAP,), jnp.int32),
            pltpu.VMEM((CH_CAP,), jnp.int32),
            pltpu.VMEM((CH_CAP,), jnp.int32),
            pltpu.VMEM((16 * 128,), jnp.float32),
            pltpu.SMEM((1,), jnp.int32),
            pltpu.SemaphoreType.DMA,
            pltpu.SemaphoreType.DMA,
        ],
        compiler_params=pltpu.CompilerParams(
            use_tc_tiling_on_sc=True, needs_layout_passes=False),
    )


def _bilinear_body(b_ref, g1_ref, g2_ref, w_ref, out_ref):
    g1 = g1_ref[:, :D]
    g2 = g2_ref[:, :D]
    t = jnp.dot(g1, w_ref[...], preferred_element_type=jnp.float32)
    p = jnp.sum(t * g2, axis=1) + b_ref[0]
    out_ref[...] = p.reshape(out_ref.shape)


@functools.lru_cache(maxsize=None)
def _make_bilinear(batch, blk):
    nblk = batch // blk
    return pl.pallas_call(
        _bilinear_body,
        grid=(nblk,),
        in_specs=[
            pl.BlockSpec(memory_space=pltpu.SMEM),
            pl.BlockSpec((blk, 128), lambda i: (i, 0)),
            pl.BlockSpec((blk, 128), lambda i, _n=nblk: (i + _n, 0)),
            pl.BlockSpec((D, D), lambda i: (0, 0)),
        ],
        out_specs=pl.BlockSpec((1, 1, blk), lambda i: (i, 0, 0)),
        out_shape=jax.ShapeDtypeStruct((nblk, 1, blk), jnp.float32),
    )


def kernel(indice_pairs, Z, W, b):
    batch = indice_pairs.shape[0]
    total = 2 * batch
    idx_flat = jnp.transpose(indice_pairs).reshape(total).astype(jnp.int32)
    g_flat = _make_sc_scan(total)(Z.T, idx_flat)
    g = g_flat.reshape(total, 128)
    blk = 2048
    pred = _make_bilinear(batch, blk)(b.reshape(1), g, g, W)
    return pred.reshape(batch)


# speedup vs baseline: 1.0187x; 1.0187x over previous
import functools

import jax
import jax.numpy as jnp
from jax import lax
from jax.experimental import pallas as pl
from jax.experimental.pallas import tpu as pltpu
from jax.experimental.pallas import tpu_sc as plsc

D = 64
NC = 2
NS = 16
NW = NC * NS
N_NODES = 1000000
CW = 512
N_FULL = N_NODES // CW
TAIL_LO = N_FULL * CW
TAIL_W = N_NODES - TAIL_LO
CHUNKS_BASE = N_FULL // NW
CHUNKS_EXTRA = N_FULL - CHUNKS_BASE * NW
IDX_PIECE = 8192
MAXJJ = (CHUNKS_BASE + CHUNKS_EXTRA + 2) // 3
BKT_CAP = 2080
CH_CAP = 560


def _iota16():
    return lax.iota(jnp.int32, 16)


def _in_range(vec, lo, hi):
    ge = lax.min(lax.max(vec - lo + 1, 0), 1)
    lt = lax.min(lax.max(hi - vec, 0), 1)
    return ge * lt


def _compact_append(ref_a, ref_b, va, vb, mi, cnt, trash):
    pos = plsc.cumsum(mi)
    dst = (cnt + pos - 1) * mi + (trash + _iota16()) * (1 - mi)
    plsc.store_scatter(ref_a, [dst], va)
    plsc.store_scatter(ref_b, [dst], vb)
    return cnt + pos[15]


def _sc_scan_fn(total):
    n_vec = total // 16

    def body(tbl, idx_hbm, out_hbm, idx_v, buf0, buf1, buf2, tail_v,
             bkt_r, bkt_s, chr_v, chs_v, stage, hg_s, semc, semo):
        wid = lax.axis_index("s") * NC + lax.axis_index("c")
        is0 = 1 - lax.min(wid, 1)
        nch = CHUNKS_BASE + CHUNKS_EXTRA * is0
        ch0 = wid * CHUNKS_BASE + CHUNKS_EXTRA * (1 - is0)
        c_lo = ch0 * CW
        c_hi = c_lo + nch * CW + TAIL_W * lax.max(wid - (NW - 2), 0)

        pltpu.async_copy(tbl.at[:, pl.ds(c_lo, CW)], buf0, semc)
        pltpu.async_copy(tbl.at[:, pl.ds(c_lo + CW, CW)], buf1, semc)
        pltpu.async_copy(tbl.at[:, pl.ds(c_lo + 2 * CW, CW)], buf2, semc)

        hg_s[0] = 0

        cnt0 = jnp.int32(0)
        for piece in range(total // IDX_PIECE):
            pltpu.sync_copy(idx_hbm.at[pl.ds(piece * IDX_PIECE, IDX_PIECE)],
                            idx_v)

            def p1(g, cnt, _piece=piece):
                vec = idx_v[pl.ds(g * 16, 16)]
                mi = _in_range(vec, c_lo, c_hi)
                slots = _iota16() + (_piece * IDX_PIECE + g * 16)
                return _compact_append(bkt_r, bkt_s, vec, slots, mi, cnt,
                                       BKT_CAP - 16)

            cnt0 = lax.fori_loop(0, IDX_PIECE // 16, p1, cnt0)
        nb = cnt0
        nbv = lax.shift_right_logical(nb + 15, 4)

        def extract_hits(lo, buf, ccnt):
            def exhit(i, carry):
                r = chr_v[pl.ds(i, 16)][0]
                slot = chs_v[pl.ds(i, 16)][0]
                rr = r - lo
                hg = hg_s[0]
                ring = lax.bitwise_and(hg, jnp.int32(15))

                @pl.when(hg >= 16)
                def _():
                    pltpu.make_async_copy(
                        out_hbm.at[pl.ds(0, 128)],
                        stage.at[pl.ds(0, 128)], semo).wait()

                for grp in range(4):
                    val = plsc.load_gather(
                        buf, [_iota16() + grp * 16,
                              jnp.broadcast_to(rr, (16,))])
                    stage[pl.ds(ring * 128 + grp * 16, 16)] = val
                pltpu.async_copy(stage.at[pl.ds(ring * 128, 128)],
                                 out_hbm.at[pl.ds(slot * 128, 128)], semo)
                hg_s[0] = hg + 1
                return carry

            lax.fori_loop(0, ccnt, exhit, jnp.int32(0))

        def scan_bucket(lo, width):
            def sv(v, ccnt):
                vecr = bkt_r[pl.ds(v * 16, 16)]
                vecs = bkt_s[pl.ds(v * 16, 16)]
                mi = _in_range(vecr, lo, lo + width)
                mi = mi * _in_range(_iota16() + v * 16, 0, nb)
                return _compact_append(chr_v, chs_v, vecr, vecs, mi, ccnt,
                                       CH_CAP - 16)

            return lax.fori_loop(0, nbv, sv, jnp.int32(0))

        def process(j, buf):
            lo = c_lo + j * CW
            pltpu.make_async_copy(tbl.at[:, pl.ds(lo, CW)], buf, semc).wait()
            ccnt = scan_bucket(lo, CW)
            extract_hits(lo, buf, ccnt)

            @pl.when(j + 3 < nch)
            def _():
                pltpu.async_copy(tbl.at[:, pl.ds(lo + 3 * CW, CW)], buf, semc)

        def jj_body(jj, carry):
            j0 = jj * 3

            @pl.when(j0 < nch)
            def _():
                process(j0, buf0)

            @pl.when(j0 + 1 < nch)
            def _():
                process(j0 + 1, buf1)

            @pl.when(j0 + 2 < nch)
            def _():
                process(j0 + 2, buf2)

            return carry

        lax.fori_loop(0, MAXJJ, jj_body, jnp.int32(0))

        @pl.when(wid == NW - 1)
        def _():
            pltpu.sync_copy(tbl.at[:, pl.ds(TAIL_LO, TAIL_W)], tail_v)
            ccnt = scan_bucket(jnp.int32(TAIL_LO), jnp.int32(TAIL_W))
            extract_hits(jnp.int32(TAIL_LO), tail_v, ccnt)

        def drain(i, carry):
            pltpu.make_async_copy(out_hbm.at[pl.ds(0, 128)],
                                  stage.at[pl.ds(0, 128)], semo).wait()
            return carry

        lax.fori_loop(0, lax.min(hg_s[0], jnp.int32(16)), drain, jnp.int32(0))

    return body


@functools.lru_cache(maxsize=None)
def _make_sc_scan(total):
    mesh = plsc.VectorSubcoreMesh(core_axis_name="c", subcore_axis_name="s")
    return pl.kernel(
        _sc_scan_fn(total),
        mesh=mesh,
        out_type=jax.ShapeDtypeStruct((total * 128,), jnp.float32),
        scratch_types=[
            pltpu.VMEM((IDX_PIECE,), jnp.int32),
            pltpu.VMEM((D, CW), jnp.float32),
            pltpu.VMEM((D, CW), jnp.float32),
            pltpu.VMEM((D, CW), jnp.float32),
            pltpu.VMEM((D, TAIL_W), jnp.float32),
            pltpu.VMEM((BKT_CAP,), jnp.int32),
            pltpu.VMEM((BKT_CAP,), jnp.int32),
            pltpu.VMEM((CH_CAP,), jnp.int32),
            pltpu.VMEM((CH_CAP,), jnp.int32),
            pltpu.VMEM((16 * 128,), jnp.float32),
            pltpu.SMEM((1,), jnp.int32),
            pltpu.SemaphoreType.DMA,
            pltpu.SemaphoreType.DMA,
        ],
        compiler_params=pltpu.CompilerParams(
            use_tc_tiling_on_sc=True, needs_layout_passes=False),
    )


def _bilinear_body(b_ref, g1_ref, g2_ref, w_ref, out_ref):
    g1 = g1_ref[:, :D]
    g2 = g2_ref[:, :D]
    t = jnp.dot(g1, w_ref[...], preferred_element_type=jnp.float32)
    p = jnp.sum(t * g2, axis=1) + b_ref[0]
    out_ref[...] = p.reshape(out_ref.shape)


@functools.lru_cache(maxsize=None)
def _make_bilinear(batch, blk):
    nblk = batch // blk
    return pl.pallas_call(
        _bilinear_body,
        grid=(nblk,),
        in_specs=[
            pl.BlockSpec(memory_space=pltpu.SMEM),
            pl.BlockSpec((blk, 128), lambda i: (i, 0)),
            pl.BlockSpec((blk, 128), lambda i, _n=nblk: (i + _n, 0)),
            pl.BlockSpec((D, D), lambda i: (0, 0)),
        ],
        out_specs=pl.BlockSpec((1, 1, blk), lambda i: (i, 0, 0)),
        out_shape=jax.ShapeDtypeStruct((nblk, 1, blk), jnp.float32),
    )


def kernel(indice_pairs, Z, W, b):
    batch = indice_pairs.shape[0]
    total = 2 * batch
    idx_flat = jnp.transpose(indice_pairs).reshape(total).astype(jnp.int32)
    g_flat = _make_sc_scan(total)(Z.T, idx_flat)
    g = g_flat.reshape(total, 128)
    blk = 4096
    pred = _make_bilinear(batch, blk)(b.reshape(1), g, g, W)
    return pred.reshape(batch)
